# baseline (speedup 1.0000x reference)
import functools

import jax
import jax.numpy as jnp
from jax import lax
from jax.experimental import pallas as pl
from jax.experimental.pallas import tpu as pltpu
from jax.experimental.pallas import tpu_sc as plsc

DEPTH = 1000
BATCH = 16384

_info = plsc.get_sparse_core_info()
_NC, _NS, _L = _info.num_cores, _info.num_subcores, _info.num_lanes
_NW = _NC * _NS
_ROWS_PER_W = BATCH // _NW
_CHUNK_ROWS = 32
_N_CHUNKS = _ROWS_PER_W // _CHUNK_ROWS
_GROUPS = _CHUNK_ROWS // _L
_FULL_COLS = DEPTH // _L


@functools.partial(
    pl.kernel,
    out_type=jax.ShapeDtypeStruct((BATCH, DEPTH), jnp.float32),
    mesh=plsc.VectorSubcoreMesh(core_axis_name="c", subcore_axis_name="s"),
    compiler_params=pltpu.CompilerParams(
        needs_layout_passes=False, use_tc_tiling_on_sc=True),
    scratch_types=[
        pltpu.VMEM((_ROWS_PER_W,), jnp.int32),
        pltpu.VMEM((_CHUNK_ROWS, DEPTH), jnp.float32),
        pltpu.VMEM((_CHUNK_ROWS, DEPTH), jnp.float32),
        pltpu.SemaphoreType.DMA,
        pltpu.SemaphoreType.DMA,
    ],
)
def _sc_onehot(idx_hbm, out_hbm, idx_v, buf0, buf1, sem0, sem1):
    wid = lax.axis_index("s") * _NC + lax.axis_index("c")
    row0 = wid * _ROWS_PER_W

    pltpu.sync_copy(idx_hbm.at[pl.ds(row0, _ROWS_PER_W)], idx_v)

    zero16 = jnp.zeros((_L,), jnp.float32)
    one16 = jnp.full((_L,), 1.0, jnp.float32)
    lanes = lax.iota(jnp.int32, _L)

    def zbody(r, carry):
        for k in range(_FULL_COLS):
            buf0[r, pl.ds(k * _L, _L)] = zero16
            buf1[r, pl.ds(k * _L, _L)] = zero16
        buf0[r, pl.ds(DEPTH - _L, _L)] = zero16
        buf1[r, pl.ds(DEPTH - _L, _L)] = zero16
        return carry

    lax.fori_loop(0, _CHUNK_ROWS, zbody, 0)

    def chunk_cols(c, g):
        return idx_v[pl.ds(c * _CHUNK_ROWS + g * _L, _L)]

    def out_slice(c):
        return out_hbm.at[pl.ds(row0 + c * _CHUNK_ROWS, _CHUNK_ROWS)]

    bufs = (buf0, buf1)
    sems = (sem0, sem1)

    for b in range(2):
        for g in range(_GROUPS):
            plsc.store_scatter(bufs[b], [g * _L + lanes, chunk_cols(b, g)], one16)
        pltpu.async_copy(bufs[b], out_slice(b), sems[b])

    def lbody(p, carry):
        for b in range(2):
            c = 2 * p + 2 + b
            pltpu.make_async_copy(bufs[b], out_slice(c - 2), sems[b]).wait()
            for g in range(_GROUPS):
                plsc.store_scatter(
                    bufs[b], [g * _L + lanes, chunk_cols(c - 2, g)], zero16)
            for g in range(_GROUPS):
                plsc.store_scatter(
                    bufs[b], [g * _L + lanes, chunk_cols(c, g)], one16)
            pltpu.async_copy(bufs[b], out_slice(c), sems[b])
        return carry

    lax.fori_loop(0, (_N_CHUNKS - 2) // 2, lbody, 0)

    pltpu.make_async_copy(buf0, out_slice(_N_CHUNKS - 2), sem0).wait()
    pltpu.make_async_copy(buf1, out_slice(_N_CHUNKS - 1), sem1).wait()


@jax.jit
def kernel(X_in, ones):
    del ones
    return _sc_onehot(X_in.astype(jnp.int32))


# speedup vs baseline: 2.2148x; 2.2148x over previous
import functools

import jax
import jax.numpy as jnp
from jax import lax
from jax.experimental import pallas as pl
from jax.experimental.pallas import tpu as pltpu
from jax.experimental.pallas import tpu_sc as plsc

DEPTH = 1000
BATCH = 16384

_info = plsc.get_sparse_core_info()
_NC, _NS, _L = _info.num_cores, _info.num_subcores, _info.num_lanes
_NW = _NC * _NS
_COLS_PER_W = BATCH // _NW
_CHUNK_COLS = 128
_N_CHUNKS = _COLS_PER_W // _CHUNK_COLS
_GROUPS = _CHUNK_COLS // _L
_ROWS_A = 512
_ROWS_B = DEPTH - _ROWS_A


@functools.partial(
    pl.kernel,
    out_type=jax.ShapeDtypeStruct((DEPTH, BATCH), jnp.float32),
    mesh=plsc.VectorSubcoreMesh(core_axis_name="c", subcore_axis_name="s"),
    compiler_params=pltpu.CompilerParams(
        needs_layout_passes=False, use_tc_tiling_on_sc=True),
    scratch_types=[
        pltpu.VMEM((_COLS_PER_W,), jnp.int32),
        pltpu.VMEM((_ROWS_A, _CHUNK_COLS), jnp.float32),
        pltpu.VMEM((_ROWS_B, _CHUNK_COLS), jnp.float32),
        pltpu.SemaphoreType.DMA,
        pltpu.SemaphoreType.DMA,
    ],
)
def _sc_onehot_t(idx_hbm, out_hbm, idx_v, buf_a, buf_b, sem_a, sem_b):
    wid = lax.axis_index("s") * _NC + lax.axis_index("c")
    col0 = wid * _COLS_PER_W

    pltpu.sync_copy(idx_hbm.at[pl.ds(col0, _COLS_PER_W)], idx_v)

    zero16 = jnp.zeros((_L,), jnp.float32)
    one16 = jnp.full((_L,), 1.0, jnp.float32)
    lanes = lax.iota(jnp.int32, _L)

    def zbody(r, carry):
        for k in range(_GROUPS):
            buf_a[r, pl.ds(k * _L, _L)] = zero16
        return carry

    def zbody_b(r, carry):
        for k in range(_GROUPS):
            buf_b[r, pl.ds(k * _L, _L)] = zero16
        return carry

    lax.fori_loop(0, _ROWS_A, zbody, 0)
    lax.fori_loop(0, _ROWS_B, zbody_b, 0)

    def scatter_chunk(c, val16):
        for g in range(_GROUPS):
            idxv = idx_v[pl.ds(c * _CHUNK_COLS + g * _L, _L)]
            colv = g * _L + lanes
            in_a = idxv < _ROWS_A
            plsc.store_scatter(buf_a, [idxv, colv], val16, mask=in_a)
            plsc.store_scatter(
                buf_b, [idxv - _ROWS_A, colv], val16,
                mask=jnp.logical_not(in_a))

    def dma_pair(c):
        cbase = col0 + c * _CHUNK_COLS
        cp_a = pltpu.make_async_copy(
            buf_a, out_hbm.at[pl.ds(0, _ROWS_A), pl.ds(cbase, _CHUNK_COLS)],
            sem_a)
        cp_b = pltpu.make_async_copy(
            buf_b,
            out_hbm.at[pl.ds(_ROWS_A, _ROWS_B), pl.ds(cbase, _CHUNK_COLS)],
            sem_b)
        return cp_a, cp_b

    def cbody(c, carry):
        scatter_chunk(c, one16)
        cp_a, cp_b = dma_pair(c)
        cp_a.start()
        cp_b.start()
        cp_a.wait()
        cp_b.wait()
        scatter_chunk(c, zero16)
        return carry

    lax.fori_loop(0, _N_CHUNKS, cbody, 0)


@jax.jit
def kernel(X_in, ones):
    del ones
    return _sc_onehot_t(X_in.astype(jnp.int32)).T
